# baseline (speedup 1.0000x reference)
import functools

import jax
import jax.numpy as jnp
from jax import lax
from jax.experimental import pallas as pl
from jax.experimental.pallas import tpu as pltpu
from jax.experimental.pallas import tpu_sc as plsc

N_RAYS = 131072
N_PTS = 64
N_OUT = 128
EPS = 1e-5
NW = 32
RAYS_PER_W = N_RAYS // NW
CH = 128
N_CHUNKS = RAYS_PER_W // CH


def _rev(x):
    return lax.rev(x, (0,))


_GDN = lax.GatherDimensionNumbers(
    offset_dims=(), collapsed_slice_dims=(0,), start_index_map=(0,))


def _gather16(x, idx):
    return lax.gather(x, idx[:, None], _GDN, slice_sizes=(1,),
                      mode=lax.GatherScatterMode.PROMISE_IN_BOUNDS)


def _prefix16(x, ii):
    for k in (1, 2, 4, 8):
        shifted = _gather16(x, jnp.maximum(ii - k, 0))
        x = x + jnp.where(ii >= k, shifted, 0.0)
    return x


def _m16(x, y):
    yr = _rev(y)
    return [jnp.sort(jnp.minimum(x, yr)), jnp.sort(jnp.maximum(x, yr))]


def _bit32(l0, l1):
    return [jnp.sort(jnp.minimum(l0, l1)), jnp.sort(jnp.maximum(l0, l1))]


def _m32(a, b):
    rb0, rb1 = _rev(b[1]), _rev(b[0])
    return (_bit32(jnp.minimum(a[0], rb0), jnp.minimum(a[1], rb1))
            + _bit32(jnp.maximum(a[0], rb0), jnp.maximum(a[1], rb1)))


def _bit64(p):
    q0, q1 = jnp.minimum(p[0], p[2]), jnp.minimum(p[1], p[3])
    q2, q3 = jnp.maximum(p[0], p[2]), jnp.maximum(p[1], p[3])
    return _bit32(q0, q1) + _bit32(q2, q3)


def _m64(a, b):
    rb = [_rev(b[3]), _rev(b[2]), _rev(b[1]), _rev(b[0])]
    l = [jnp.minimum(a[i], rb[i]) for i in range(4)]
    h = [jnp.maximum(a[i], rb[i]) for i in range(4)]
    return _bit64(l) + _bit64(h)


def _sort128(v):
    s = [jnp.sort(x) for x in v]
    a = _m32(_m16(s[0], s[1]), _m16(s[2], s[3]))
    b = _m32(_m16(s[4], s[5]), _m16(s[6], s[7]))
    return _m64(a, b)


_mesh = plsc.VectorSubcoreMesh(core_axis_name="c", subcore_axis_name="s")


@functools.partial(
    pl.kernel,
    mesh=_mesh,
    out_type=jax.ShapeDtypeStruct((N_RAYS, N_OUT), jnp.float32),
    scratch_types=[
        pltpu.VMEM((CH, N_PTS), jnp.float32),
        pltpu.VMEM((CH, N_PTS), jnp.float32),
        pltpu.VMEM((CH, N_OUT), jnp.float32),
        pltpu.VMEM((CH, N_PTS), jnp.float32),
        pltpu.VMEM((CH, N_PTS), jnp.float32),
        pltpu.VMEM((CH, N_OUT), jnp.float32),
        pltpu.VMEM((CH * N_PTS,), jnp.float32),
        pltpu.VMEM((CH * N_PTS,), jnp.float32),
        pltpu.VMEM((CH * N_PTS,), jnp.int32),
        pltpu.VMEM((64,), jnp.float32),
        pltpu.SemaphoreType.DMA,
        pltpu.SemaphoreType.DMA,
        pltpu.SemaphoreType.DMA,
        pltpu.SemaphoreType.DMA,
    ],
    compiler_params=pltpu.CompilerParams(needs_layout_passes=False),
)
def _refine(z_hbm, w_hbm, u_hbm, out_hbm, zb0, wb0, ob0, zb1, wb1, ob1,
            cb, bb, hb, ub, si0, si1, so0, so1):
    wid = lax.axis_index("s") * 2 + lax.axis_index("c")
    pltpu.sync_copy(u_hbm, ub)
    ii = lax.iota(jnp.int32, 16)
    zero = jnp.zeros((16,), jnp.float32)
    u_vecs = [ub[pl.ds(16 * g, 16)] for g in range(4)]
    zbs, wbs, obs = (zb0, zb1), (wb0, wb1), (ob0, ob1)
    sins, souts = (si0, si1), (so0, so1)

    def hbase(c):
        return wid * RAYS_PER_W + c * CH

    def in_copies(c, b):
        return (pltpu.make_async_copy(z_hbm.at[pl.ds(hbase(c), CH)], zbs[b], sins[b]),
                pltpu.make_async_copy(w_hbm.at[pl.ds(hbase(c), CH)], wbs[b], sins[b]))

    def out_copy(c, b):
        return pltpu.make_async_copy(obs[b], out_hbm.at[pl.ds(hbase(c), CH)], souts[b])

    def start_in(c, b):
        for cp in in_copies(c, b):
            cp.start()

    for cp in in_copies(0, 0):
        cp.start()

    def pair_body(i, carry):
        c0 = i * 2
        for b in range(2):
            c = c0 + b
            compute(c, b)
        return carry

    def compute(c, b):
        zb, wb, ob = zbs[b], wbs[b], obs[b]

        @pl.when(c + 1 < N_CHUNKS)
        def _():
            start_in(c + 1, 1 - b)

        for cp in in_copies(c, b):
            cp.wait()

        @pl.when(c >= 2)
        def _():
            out_copy(c, b).wait()

        @plsc.parallel_loop(0, CH, 1, unroll=2)
        def ray_body(r):
            r64 = r * N_PTS
            z = [zb[r, pl.ds(16 * g, 16)] for g in range(4)]
            w = [wb[r, pl.ds(16 * g, 16)] for g in range(4)]
            w0 = jnp.where(ii >= 1, w[0] + EPS, 0.0)
            w1 = w[1] + EPS
            w2 = w[2] + EPS
            w3 = jnp.where(ii <= 14, w[3] + EPS, 0.0)
            c0 = plsc.cumsum(w0)
            c1 = plsc.cumsum(w1) + c0[15]
            c2 = plsc.cumsum(w2) + c1[15]
            c3 = plsc.cumsum(w3) + c2[15]
            s_tot = c3[15]
            rs = 1.0 / jnp.full((16,), s_tot, jnp.float32)
            cb[pl.ds(r64, 16)] = c0
            cb[pl.ds(r64 + 16, 16)] = c1
            cb[pl.ds(r64 + 32, 16)] = c2
            cb[pl.ds(r64 + 48, 16)] = jnp.where(ii >= 15, jnp.inf, c3)
            fr = jnp.full((16,), r, jnp.int32)
            for g in range(4):
                idx = ii + (16 * g + 1)
                if g == 3:
                    idx = jnp.minimum(idx, 63)
                zk1 = plsc.load_gather(zb, [fr, idx])
                bb[pl.ds(r64 + 16 * g, 16)] = zk1 + 0.5 * (z[g] - zk1)
            base = jnp.full((16,), r64, jnp.int32)
            izero = jnp.zeros((16,), jnp.int32)
            ione = jnp.ones((16,), jnp.int32)
            rs63 = rs * 63.0
            for g in range(4):
                hb[pl.ds(r64 + 16 * g, 16)] = izero
            for g, cg in enumerate((c0, c1, c2, c3)):
                y = cg * rs63
                ti = y.astype(jnp.int32)
                n = ti + (ti.astype(jnp.float32) < y).astype(jnp.int32)
                mask = n <= 63
                if g == 0:
                    mask = mask & (ii >= 1)
                if g == 3:
                    mask = mask & (ii <= 14)
                plsc.addupdate_scatter(hb, [base + n], ione, mask=mask)
            p0 = plsc.cumsum(hb[pl.ds(r64, 16)])
            p1 = plsc.cumsum(hb[pl.ds(r64 + 16, 16)]) + p0[15]
            p2 = plsc.cumsum(hb[pl.ds(r64 + 32, 16)]) + p1[15]
            p3 = plsc.cumsum(hb[pl.ds(r64 + 48, 16)]) + p2[15]
            ms = (p0 + base, p1 + base, p2 + base, p3 + base)
            smp = []
            for g in range(4):
                t = u_vecs[g] * s_tot
                m = ms[g]
                na = m + 1
                vm = plsc.load_gather(cb, [m])
                cg1 = plsc.load_gather(cb, [na])
                bg0 = plsc.load_gather(bb, [m])
                bg1 = plsc.load_gather(bb, [na])
                dncdf = (cg1 - vm) * rs
                tf = (t - vm) * rs
                tfrac = jnp.where(dncdf < EPS, tf, tf / dncdf)
                smp.append(bg0 + tfrac * (bg1 - bg0))
            o = _sort128(z + smp)
            for g in range(8):
                ob[r, pl.ds(16 * g, 16)] = o[g]

        out_copy(c, b).start()

    lax.fori_loop(0, N_CHUNKS // 2, pair_body, 0)
    out_copy(N_CHUNKS - 2, 0).wait()
    out_copy(N_CHUNKS - 1, 1).wait()


def kernel(lengths, ray_weights):
    u = jnp.linspace(0.0, 1.0, N_PTS, dtype=jnp.float32)
    return _refine(lengths, ray_weights, u)


# speedup vs baseline: 1.0927x; 1.0927x over previous
import functools

import jax
import jax.numpy as jnp
from jax import lax
from jax.experimental import pallas as pl
from jax.experimental.pallas import tpu as pltpu
from jax.experimental.pallas import tpu_sc as plsc

N_RAYS = 131072
N_PTS = 64
N_OUT = 128
EPS = 1e-5
NW = 32
RAYS_PER_W = N_RAYS // NW
CH = 128
N_CHUNKS = RAYS_PER_W // CH


def _rev(x):
    return lax.rev(x, (0,))


_GDN = lax.GatherDimensionNumbers(
    offset_dims=(), collapsed_slice_dims=(0,), start_index_map=(0,))


def _gather16(x, idx):
    return lax.gather(x, idx[:, None], _GDN, slice_sizes=(1,),
                      mode=lax.GatherScatterMode.PROMISE_IN_BOUNDS)


def _prefix16(x, ii):
    for k in (1, 2, 4, 8):
        shifted = _gather16(x, jnp.maximum(ii - k, 0))
        x = x + jnp.where(ii >= k, shifted, 0.0)
    return x


def _vsort(x, desc):
    if desc:
        r = plsc.sort_key_val(x, x, descending=True)
        return r[0] if isinstance(r, (list, tuple)) else r
    return jnp.sort(x)


def _m16(x, y, desc):
    lo, hi = jnp.minimum(x, y), jnp.maximum(x, y)
    if desc:
        return [_vsort(hi, True), _vsort(lo, True)]
    return [_vsort(lo, False), _vsort(hi, False)]


def _bit32(p, desc):
    lo, hi = jnp.minimum(p[0], p[1]), jnp.maximum(p[0], p[1])
    if desc:
        return [_vsort(hi, True), _vsort(lo, True)]
    return [_vsort(lo, False), _vsort(hi, False)]


def _m32(a, b, desc):
    l = [jnp.minimum(a[i], b[i]) for i in range(2)]
    h = [jnp.maximum(a[i], b[i]) for i in range(2)]
    if desc:
        return _bit32(h, True) + _bit32(l, True)
    return _bit32(l, False) + _bit32(h, False)


def _bit64(p, desc):
    l = [jnp.minimum(p[i], p[i + 2]) for i in range(2)]
    h = [jnp.maximum(p[i], p[i + 2]) for i in range(2)]
    if desc:
        return _bit32(h, True) + _bit32(l, True)
    return _bit32(l, False) + _bit32(h, False)


def _m64(a, b):
    l = [jnp.minimum(a[i], b[i]) for i in range(4)]
    h = [jnp.maximum(a[i], b[i]) for i in range(4)]
    return _bit64(l, False) + _bit64(h, False)


def _sort128(v):
    a = _m32(_m16(_vsort(v[0], False), _vsort(v[1], True), False),
             _m16(_vsort(v[2], False), _vsort(v[3], True), True), False)
    b = _m32(_m16(_vsort(v[4], False), _vsort(v[5], True), False),
             _m16(_vsort(v[6], False), _vsort(v[7], True), True), True)
    return _m64(a, b)


_mesh = plsc.VectorSubcoreMesh(core_axis_name="c", subcore_axis_name="s")


@functools.partial(
    pl.kernel,
    mesh=_mesh,
    out_type=jax.ShapeDtypeStruct((N_RAYS, N_OUT), jnp.float32),
    scratch_types=[
        pltpu.VMEM((CH, N_PTS), jnp.float32),
        pltpu.VMEM((CH, N_PTS), jnp.float32),
        pltpu.VMEM((CH, N_OUT), jnp.float32),
        pltpu.VMEM((CH, N_PTS), jnp.float32),
        pltpu.VMEM((CH, N_PTS), jnp.float32),
        pltpu.VMEM((CH, N_OUT), jnp.float32),
        pltpu.VMEM((CH * N_PTS,), jnp.float32),
        pltpu.VMEM((CH * N_PTS,), jnp.float32),
        pltpu.VMEM((CH * N_PTS,), jnp.int32),
        pltpu.VMEM((64,), jnp.float32),
        pltpu.SemaphoreType.DMA,
        pltpu.SemaphoreType.DMA,
        pltpu.SemaphoreType.DMA,
        pltpu.SemaphoreType.DMA,
    ],
    compiler_params=pltpu.CompilerParams(needs_layout_passes=False),
)
def _refine(z_hbm, w_hbm, u_hbm, out_hbm, zb0, wb0, ob0, zb1, wb1, ob1,
            cb, bb, hb, ub, si0, si1, so0, so1):
    wid = lax.axis_index("s") * 2 + lax.axis_index("c")
    pltpu.sync_copy(u_hbm, ub)
    ii = lax.iota(jnp.int32, 16)
    zero = jnp.zeros((16,), jnp.float32)
    u_vecs = [ub[pl.ds(16 * g, 16)] for g in range(4)]
    zbs, wbs, obs = (zb0, zb1), (wb0, wb1), (ob0, ob1)
    sins, souts = (si0, si1), (so0, so1)

    def hbase(c):
        return wid * RAYS_PER_W + c * CH

    def in_copies(c, b):
        return (pltpu.make_async_copy(z_hbm.at[pl.ds(hbase(c), CH)], zbs[b], sins[b]),
                pltpu.make_async_copy(w_hbm.at[pl.ds(hbase(c), CH)], wbs[b], sins[b]))

    def out_copy(c, b):
        return pltpu.make_async_copy(obs[b], out_hbm.at[pl.ds(hbase(c), CH)], souts[b])

    def start_in(c, b):
        for cp in in_copies(c, b):
            cp.start()

    for cp in in_copies(0, 0):
        cp.start()

    def pair_body(i, carry):
        c0 = i * 2
        for b in range(2):
            c = c0 + b
            compute(c, b)
        return carry

    def compute(c, b):
        zb, wb, ob = zbs[b], wbs[b], obs[b]

        @pl.when(c + 1 < N_CHUNKS)
        def _():
            start_in(c + 1, 1 - b)

        for cp in in_copies(c, b):
            cp.wait()

        @pl.when(c >= 2)
        def _():
            out_copy(c, b).wait()

        @plsc.parallel_loop(0, CH, 1, unroll=2)
        def ray_body(r):
            r64 = r * N_PTS
            z = [zb[r, pl.ds(16 * g, 16)] for g in range(4)]
            w = [wb[r, pl.ds(16 * g, 16)] for g in range(4)]
            w0 = jnp.where(ii >= 1, w[0] + EPS, 0.0)
            w1 = w[1] + EPS
            w2 = w[2] + EPS
            w3 = jnp.where(ii <= 14, w[3] + EPS, 0.0)
            c0 = plsc.cumsum(w0)
            c1 = plsc.cumsum(w1) + c0[15]
            c2 = plsc.cumsum(w2) + c1[15]
            c3 = plsc.cumsum(w3) + c2[15]
            s_tot = c3[15]
            rs = 1.0 / jnp.full((16,), s_tot, jnp.float32)
            cb[pl.ds(r64, 16)] = c0
            cb[pl.ds(r64 + 16, 16)] = c1
            cb[pl.ds(r64 + 32, 16)] = c2
            cb[pl.ds(r64 + 48, 16)] = jnp.where(ii >= 15, jnp.inf, c3)
            shift_idx = jnp.minimum(ii + 1, 15)
            for g in range(4):
                zk1 = _gather16(z[g], shift_idx)
                if g < 3:
                    zk1 = jnp.where(ii >= 15, z[g + 1][0], zk1)
                bb[pl.ds(r64 + 16 * g, 16)] = zk1 + 0.5 * (z[g] - zk1)
            base = jnp.full((16,), r64, jnp.int32)
            izero = jnp.zeros((16,), jnp.int32)
            ione = jnp.ones((16,), jnp.int32)
            rs63 = rs * 63.0
            for g in range(4):
                hb[pl.ds(r64 + 16 * g, 16)] = izero
            for g, cg in enumerate((c0, c1, c2, c3)):
                y = cg * rs63
                ti = y.astype(jnp.int32)
                n = ti + (ti.astype(jnp.float32) < y).astype(jnp.int32)
                mask = n <= 63
                if g == 0:
                    mask = mask & (ii >= 1)
                if g == 3:
                    mask = mask & (ii <= 14)
                plsc.addupdate_scatter(hb, [base + n], ione, mask=mask)
            p0 = plsc.cumsum(hb[pl.ds(r64, 16)])
            p1 = plsc.cumsum(hb[pl.ds(r64 + 16, 16)]) + p0[15]
            p2 = plsc.cumsum(hb[pl.ds(r64 + 32, 16)]) + p1[15]
            p3 = plsc.cumsum(hb[pl.ds(r64 + 48, 16)]) + p2[15]
            ms = (p0 + base, p1 + base, p2 + base, p3 + base)
            smp = []
            for g in range(4):
                t = u_vecs[g] * s_tot
                m = ms[g]
                na = m + 1
                vm = plsc.load_gather(cb, [m])
                cg1 = plsc.load_gather(cb, [na])
                bg0 = plsc.load_gather(bb, [m])
                bg1 = plsc.load_gather(bb, [na])
                dncdf = (cg1 - vm) * rs
                tf = (t - vm) * rs
                tfrac = jnp.where(dncdf < EPS, tf, tf / dncdf)
                smp.append(bg0 + tfrac * (bg1 - bg0))
            o = _sort128(z + smp)
            for g in range(8):
                ob[r, pl.ds(16 * g, 16)] = o[g]

        out_copy(c, b).start()

    lax.fori_loop(0, N_CHUNKS // 2, pair_body, 0)
    out_copy(N_CHUNKS - 2, 0).wait()
    out_copy(N_CHUNKS - 1, 1).wait()


def kernel(lengths, ray_weights):
    u = jnp.linspace(0.0, 1.0, N_PTS, dtype=jnp.float32)
    return _refine(lengths, ray_weights, u)


# speedup vs baseline: 1.1278x; 1.0320x over previous
import functools

import jax
import jax.numpy as jnp
from jax import lax
from jax.experimental import pallas as pl
from jax.experimental.pallas import tpu as pltpu
from jax.experimental.pallas import tpu_sc as plsc

N_RAYS = 131072
N_PTS = 64
N_OUT = 128
EPS = 1e-5
NW = 32
RAYS_PER_W = N_RAYS // NW
CH = 128
N_CHUNKS = RAYS_PER_W // CH


def _rev(x):
    return lax.rev(x, (0,))


_GDN = lax.GatherDimensionNumbers(
    offset_dims=(), collapsed_slice_dims=(0,), start_index_map=(0,))


def _gather16(x, idx):
    return lax.gather(x, idx[:, None], _GDN, slice_sizes=(1,),
                      mode=lax.GatherScatterMode.PROMISE_IN_BOUNDS)


def _m16(x, y):
    yr = _rev(y)
    return [jnp.sort(jnp.minimum(x, yr)), jnp.sort(jnp.maximum(x, yr))]


def _bit32(l0, l1):
    return [jnp.sort(jnp.minimum(l0, l1)), jnp.sort(jnp.maximum(l0, l1))]


def _m32(a, b):
    rb0, rb1 = _rev(b[1]), _rev(b[0])
    return (_bit32(jnp.minimum(a[0], rb0), jnp.minimum(a[1], rb1))
            + _bit32(jnp.maximum(a[0], rb0), jnp.maximum(a[1], rb1)))


def _bit64(p):
    q0, q1 = jnp.minimum(p[0], p[2]), jnp.minimum(p[1], p[3])
    q2, q3 = jnp.maximum(p[0], p[2]), jnp.maximum(p[1], p[3])
    return _bit32(q0, q1) + _bit32(q2, q3)


def _m64(a, b):
    rb = [_rev(b[3]), _rev(b[2]), _rev(b[1]), _rev(b[0])]
    l = [jnp.minimum(a[i], rb[i]) for i in range(4)]
    h = [jnp.maximum(a[i], rb[i]) for i in range(4)]
    return _bit64(l) + _bit64(h)


def _sort128(v):
    s = [jnp.sort(x) for x in v]
    a = _m32(_m16(s[0], s[1]), _m16(s[2], s[3]))
    b = _m32(_m16(s[4], s[5]), _m16(s[6], s[7]))
    return _m64(a, b)


_mesh = plsc.VectorSubcoreMesh(core_axis_name="c", subcore_axis_name="s")


@functools.partial(
    pl.kernel,
    mesh=_mesh,
    out_type=jax.ShapeDtypeStruct((N_RAYS, N_OUT), jnp.float32),
    scratch_types=[
        pltpu.VMEM((CH, N_PTS), jnp.float32),
        pltpu.VMEM((CH, N_PTS), jnp.float32),
        pltpu.VMEM((CH, N_OUT), jnp.float32),
        pltpu.VMEM((CH, N_PTS), jnp.float32),
        pltpu.VMEM((CH, N_PTS), jnp.float32),
        pltpu.VMEM((CH, N_OUT), jnp.float32),
        pltpu.VMEM((CH * N_PTS,), jnp.float32),
        pltpu.VMEM((CH * N_PTS,), jnp.float32),
        pltpu.VMEM((CH * N_PTS,), jnp.int32),
        pltpu.VMEM((64,), jnp.float32),
        pltpu.SemaphoreType.DMA,
        pltpu.SemaphoreType.DMA,
        pltpu.SemaphoreType.DMA,
        pltpu.SemaphoreType.DMA,
    ],
    compiler_params=pltpu.CompilerParams(needs_layout_passes=False),
)
def _refine(z_hbm, w_hbm, u_hbm, out_hbm, zb0, wb0, ob0, zb1, wb1, ob1,
            cb, bb, hb, ub, si0, si1, so0, so1):
    wid = lax.axis_index("s") * 2 + lax.axis_index("c")
    pltpu.sync_copy(u_hbm, ub)
    ii = lax.iota(jnp.int32, 16)
    u_vecs = [ub[pl.ds(16 * g, 16)] for g in range(4)]
    zbs, wbs, obs = (zb0, zb1), (wb0, wb1), (ob0, ob1)
    sins, souts = (si0, si1), (so0, so1)

    def hbase(c):
        return wid * RAYS_PER_W + c * CH

    def in_copies(c, b):
        return (pltpu.make_async_copy(z_hbm.at[pl.ds(hbase(c), CH)], zbs[b], sins[b]),
                pltpu.make_async_copy(w_hbm.at[pl.ds(hbase(c), CH)], wbs[b], sins[b]))

    def out_copy(c, b):
        return pltpu.make_async_copy(obs[b], out_hbm.at[pl.ds(hbase(c), CH)], souts[b])

    def start_in(c, b):
        for cp in in_copies(c, b):
            cp.start()

    for cp in in_copies(0, 0):
        cp.start()

    def pair_body(i, carry):
        c0 = i * 2
        for b in range(2):
            c = c0 + b
            compute(c, b)
        return carry

    def compute(c, b):
        zb, wb, ob = zbs[b], wbs[b], obs[b]

        @pl.when(c + 1 < N_CHUNKS)
        def _():
            start_in(c + 1, 1 - b)

        for cp in in_copies(c, b):
            cp.wait()

        @pl.when(c >= 2)
        def _():
            out_copy(c, b).wait()

        @plsc.parallel_loop(0, CH, 1, unroll=2)
        def ray_body(r):
            r64 = r * N_PTS
            z = [zb[r, pl.ds(16 * g, 16)] for g in range(4)]
            w = [wb[r, pl.ds(16 * g, 16)] for g in range(4)]
            w0 = jnp.where(ii >= 1, w[0] + EPS, 0.0)
            w1 = w[1] + EPS
            w2 = w[2] + EPS
            w3 = jnp.where(ii <= 14, w[3] + EPS, 0.0)
            c0 = plsc.cumsum(w0)
            c1 = plsc.cumsum(w1) + c0[15]
            c2 = plsc.cumsum(w2) + c1[15]
            c3 = plsc.cumsum(w3) + c2[15]
            s_tot = c3[15]
            rs = 1.0 / jnp.full((16,), s_tot, jnp.float32)
            cb[pl.ds(r64, 16)] = c0
            cb[pl.ds(r64 + 16, 16)] = c1
            cb[pl.ds(r64 + 32, 16)] = c2
            cb[pl.ds(r64 + 48, 16)] = jnp.where(ii >= 15, jnp.inf, c3)
            shift_idx = jnp.minimum(ii + 1, 15)
            for g in range(4):
                zk1 = _gather16(z[g], shift_idx)
                if g < 3:
                    zk1 = jnp.where(ii >= 15, z[g + 1][0], zk1)
                bb[pl.ds(r64 + 16 * g, 16)] = zk1 + 0.5 * (z[g] - zk1)
            base = jnp.full((16,), r64, jnp.int32)
            izero = jnp.zeros((16,), jnp.int32)
            ione = jnp.ones((16,), jnp.int32)
            rs63 = rs * 63.0
            for g in range(4):
                hb[pl.ds(r64 + 16 * g, 16)] = izero
            for g, cg in enumerate((c0, c1, c2, c3)):
                y = cg * rs63
                ti = y.astype(jnp.int32)
                n = ti + (ti.astype(jnp.float32) < y).astype(jnp.int32)
                mask = n <= 63
                if g == 0:
                    mask = mask & (ii >= 1)
                if g == 3:
                    mask = mask & (ii <= 14)
                plsc.addupdate_scatter(hb, [base + n], ione, mask=mask)
            p0 = plsc.cumsum(hb[pl.ds(r64, 16)])
            p1 = plsc.cumsum(hb[pl.ds(r64 + 16, 16)]) + p0[15]
            p2 = plsc.cumsum(hb[pl.ds(r64 + 32, 16)]) + p1[15]
            p3 = plsc.cumsum(hb[pl.ds(r64 + 48, 16)]) + p2[15]
            ms = (p0 + base, p1 + base, p2 + base, p3 + base)
            smp = []
            for g in range(4):
                t = u_vecs[g] * s_tot
                m = ms[g]
                na = m + 1
                vm = plsc.load_gather(cb, [m])
                cg1 = plsc.load_gather(cb, [na])
                bg0 = plsc.load_gather(bb, [m])
                bg1 = plsc.load_gather(bb, [na])
                dncdf = (cg1 - vm) * rs
                tf = (t - vm) * rs
                tfrac = jnp.where(dncdf < EPS, tf, tf / dncdf)
                smp.append(bg0 + tfrac * (bg1 - bg0))
            o = _sort128(z + smp)
            for g in range(8):
                ob[r, pl.ds(16 * g, 16)] = o[g]

        out_copy(c, b).start()

    lax.fori_loop(0, N_CHUNKS // 2, pair_body, 0)
    out_copy(N_CHUNKS - 2, 0).wait()
    out_copy(N_CHUNKS - 1, 1).wait()


def kernel(lengths, ray_weights):
    u = jnp.linspace(0.0, 1.0, N_PTS, dtype=jnp.float32)
    return _refine(lengths, ray_weights, u)
